# Initial kernel scaffold; baseline (speedup 1.0000x reference)
import jax
import jax.numpy as jnp
from jax.experimental import pallas as pl


def kernel(x, ei, batch, W1, b1, W2, b2, Wl, bl):
    raise NotImplementedError("write your pallas kernel here")


# speedup vs baseline: 16.3605x; 16.3605x over previous
import functools

import jax
import jax.numpy as jnp
from jax import lax
from jax.experimental import pallas as pl
from jax.experimental.pallas import tpu as pltpu
from jax.experimental.pallas import tpu_sc as plsc

N = 10000
E = 320000
D = 128
H = 64
O = 6
G = 16

NC = 2
NS = 16
NW = NC * NS
CHUNK = 128

ET = E + N
EP = ((ET + NW * CHUNK - 1) // (NW * CHUNK)) * (NW * CHUNK)
CE = EP // NW
NCH = CE // CHUNK

NACC = 10240
RPT = NACC // NS
DUMMY = N

_mesh = plsc.VectorSubcoreMesh(core_axis_name="c", subcore_axis_name="s")
_sc_params = pltpu.CompilerParams(use_tc_tiling_on_sc=False)


def _sc_degree_body(dst_hbm, ones_hbm, zeros_hbm, out_hbm,
                    acc_sh, didx_v, ones_v, zbuf_v, gsem):
    c = lax.axis_index("c")
    s = lax.axis_index("s")
    wid = c * NS + s
    pltpu.sync_copy(zeros_hbm, zbuf_v)
    pltpu.sync_copy(zbuf_v, acc_sh.at[pl.ds(s * RPT, RPT)])
    pltpu.sync_copy(ones_hbm, ones_v)
    plsc.subcore_barrier()
    base = wid * CE

    def step(k, carry):
        off = pl.multiple_of(base + k * CHUNK, 128)
        pltpu.sync_copy(dst_hbm.at[pl.ds(off, CHUNK)], didx_v)
        pltpu.sync_copy(ones_v, acc_sh.at[didx_v], add=True)
        return carry

    lax.fori_loop(0, NCH, step, 0)
    plsc.subcore_barrier()
    out_off = pl.multiple_of(c * NACC + s * RPT, 128)
    pltpu.sync_copy(acc_sh.at[pl.ds(s * RPT, RPT)], zbuf_v)
    pltpu.sync_copy(zbuf_v, out_hbm.at[pl.ds(out_off, RPT)])


_sc_degree = pl.kernel(
    _sc_degree_body,
    out_type=jax.ShapeDtypeStruct((NC * NACC, 16), jnp.float32),
    mesh=_mesh,
    compiler_params=_sc_params,
    scratch_types=[
        pltpu.VMEM_SHARED((NACC, 16), jnp.float32),
        pltpu.VMEM((CHUNK,), jnp.int32),
        pltpu.VMEM((CHUNK, 16), jnp.float32),
        pltpu.VMEM((RPT, 16), jnp.float32),
        pltpu.SemaphoreType.DMA,
    ],
)


def _sc_agg_body(hs_hbm, src_hbm, dst_hbm, zeros_hbm, out_hbm,
                 acc_sh, sidx_v, didx_v, rows_v, zbuf_v, gsem):
    c = lax.axis_index("c")
    s = lax.axis_index("s")
    wid = c * NS + s
    pltpu.sync_copy(zeros_hbm, zbuf_v)
    pltpu.sync_copy(zbuf_v, acc_sh.at[pl.ds(s * RPT, RPT)])
    plsc.subcore_barrier()
    base = wid * CE

    def step(k, carry):
        off = pl.multiple_of(base + k * CHUNK, 128)
        pltpu.sync_copy(src_hbm.at[pl.ds(off, CHUNK)], sidx_v)
        pltpu.sync_copy(dst_hbm.at[pl.ds(off, CHUNK)], didx_v)
        pltpu.async_copy(hs_hbm.at[sidx_v], rows_v, gsem).wait()
        pltpu.sync_copy(rows_v, acc_sh.at[didx_v], add=True)
        return carry

    lax.fori_loop(0, NCH, step, 0)
    plsc.subcore_barrier()
    out_off = pl.multiple_of(c * NACC + s * RPT, 128)
    pltpu.sync_copy(acc_sh.at[pl.ds(s * RPT, RPT)], zbuf_v)
    pltpu.sync_copy(zbuf_v, out_hbm.at[pl.ds(out_off, RPT)])


_sc_aggregate = pl.kernel(
    _sc_agg_body,
    out_type=jax.ShapeDtypeStruct((NC * NACC, H), jnp.float32),
    mesh=_mesh,
    compiler_params=_sc_params,
    scratch_types=[
        pltpu.VMEM_SHARED((NACC, H), jnp.float32),
        pltpu.VMEM((CHUNK,), jnp.int32),
        pltpu.VMEM((CHUNK,), jnp.int32),
        pltpu.VMEM((CHUNK, H), jnp.float32),
        pltpu.VMEM((RPT, H), jnp.float32),
        pltpu.SemaphoreType.DMA,
    ],
)


def _dinv_from(deg_ref):
    deg = deg_ref[0:N, 0:1] + deg_ref[NACC:NACC + N, 0:1]
    return lax.rsqrt(deg)


def _tc_scale1_body(x_ref, w1_ref, deg_ref, o_ref):
    h = jnp.dot(x_ref[...], w1_ref[...], preferred_element_type=jnp.float32,
                precision=lax.Precision.HIGHEST)
    o_ref[...] = h * _dinv_from(deg_ref)


def _tc_mid_body(p_ref, deg_ref, b1_ref, w2_ref, o_ref):
    dinv = _dinv_from(deg_ref)
    agg = p_ref[0:N, :] + p_ref[NACC:NACC + N, :]
    h1 = jnp.maximum(agg * dinv + b1_ref[...], 0.0)
    hs2 = jnp.dot(h1, w2_ref[...], preferred_element_type=jnp.float32,
                  precision=lax.Precision.HIGHEST)
    o_ref[...] = hs2 * dinv


def _tc_final_body(p_ref, deg_ref, b2_ref, batch_ref, wl_ref, bl_ref, o_ref):
    dinv = _dinv_from(deg_ref)
    agg = p_ref[0:N, :] + p_ref[NACC:NACC + N, :]
    h2 = agg * dinv + b2_ref[...]
    seg = lax.broadcasted_iota(jnp.int32, (N, G), 1)
    onehot = jnp.where(batch_ref[...] == seg, 1.0, 0.0).astype(jnp.float32)
    h2x = jnp.concatenate([h2, jnp.ones((N, 1), jnp.float32)], axis=1)
    sums = lax.dot_general(onehot, h2x, (((0,), (0,)), ((), ())),
                           preferred_element_type=jnp.float32,
                           precision=lax.Precision.HIGHEST)
    cnt = sums[:, H:H + 1]
    pooled = sums[:, 0:H] / jnp.maximum(cnt, 1.0)
    o_ref[...] = jnp.dot(pooled, wl_ref[...], preferred_element_type=jnp.float32,
                         precision=lax.Precision.HIGHEST) + bl_ref[...]


_tc_scale1 = pl.pallas_call(
    _tc_scale1_body, out_shape=jax.ShapeDtypeStruct((N, H), jnp.float32))
_tc_mid = pl.pallas_call(
    _tc_mid_body, out_shape=jax.ShapeDtypeStruct((N, H), jnp.float32))
_tc_final = pl.pallas_call(
    _tc_final_body, out_shape=jax.ShapeDtypeStruct((G, O), jnp.float32))


def kernel(x, ei, batch, W1, b1, W2, b2, Wl, bl):
    loops = jnp.arange(N, dtype=jnp.int32)
    pad = EP - ET
    src = jnp.concatenate([ei[0], loops, jnp.zeros((pad,), jnp.int32)])
    dst = jnp.concatenate([ei[1], loops, jnp.full((pad,), DUMMY, jnp.int32)])

    zeros64 = jnp.zeros((RPT, H), jnp.float32)
    zeros16 = jnp.zeros((RPT, 16), jnp.float32)
    ones16 = jnp.ones((CHUNK, 16), jnp.float32)

    deg_p = _sc_degree(dst, ones16, zeros16)
    hs1 = _tc_scale1(x, W1, deg_p)
    p1 = _sc_aggregate(hs1, src, dst, zeros64)
    hs2 = _tc_mid(p1, deg_p, b1.reshape(1, H), W2)
    p2 = _sc_aggregate(hs2, src, dst, zeros64)
    return _tc_final(p2, deg_p, b2.reshape(1, H), batch.reshape(N, 1),
                     Wl, bl.reshape(1, O))
